# baseline (speedup 1.0000x reference)
import functools

import jax
import jax.numpy as jnp
from jax import lax
from jax.experimental import pallas as pl
from jax.experimental.pallas import tpu as pltpu
from jax.experimental.pallas import tpu_sc as plsc

VOCAB = 100000
EMB = 1024
HID = 4096
EPS = 1e-12

NC = 2
NS = 16
NW = NC * NS

TOK = 8192
CHUNK = 32
M_BLK = 512
SEGMENTS = ((0, 2048), (2048, 6144))


def _sc_gather_body(start, npw, ids_hbm, table_hbm, out_hbm, idx_v,
                    buf0, buf1, sg0, sg1, sc0, sc1):
    nch = npw // CHUNK
    c = lax.axis_index("c")
    s = lax.axis_index("s")
    wid = s * NC + c
    base = wid * npw
    pltpu.sync_copy(ids_hbm.at[pl.ds(start + base, npw)], idx_v)
    bufs = (buf0, buf1)
    gsem = (sg0, sg1)
    csem = (sc0, sc1)
    g = [None] * nch
    cp = [None] * nch
    g[0] = pltpu.async_copy(
        table_hbm.at[idx_v.at[pl.ds(0, CHUNK)]], bufs[0], gsem[0])
    for ch in range(nch):
        p = ch % 2
        g[ch].wait()
        if ch >= 1:
            cp[ch - 1].wait()
        if ch + 1 < nch:
            g[ch + 1] = pltpu.async_copy(
                table_hbm.at[idx_v.at[pl.ds((ch + 1) * CHUNK, CHUNK)]],
                bufs[(ch + 1) % 2], gsem[(ch + 1) % 2])
        cp[ch] = pltpu.async_copy(
            bufs[p], out_hbm.at[pl.ds(base + ch * CHUNK, CHUNK)], csem[p])
    cp[nch - 1].wait()


@functools.partial(jax.jit, static_argnums=(0, 1))
def _sc_gather(start, ntok, ids, table):
    npw = ntok // NW
    mesh = plsc.VectorSubcoreMesh(core_axis_name="c", subcore_axis_name="s")
    run = functools.partial(
        pl.kernel,
        mesh=mesh,
        out_type=jax.ShapeDtypeStruct((ntok, EMB), jnp.float32),
        scratch_types=[
            pltpu.VMEM((npw,), jnp.int32),
            pltpu.VMEM((CHUNK, EMB), jnp.float32),
            pltpu.VMEM((CHUNK, EMB), jnp.float32),
            pltpu.SemaphoreType.DMA,
            pltpu.SemaphoreType.DMA,
            pltpu.SemaphoreType.DMA,
            pltpu.SemaphoreType.DMA,
        ],
    )(functools.partial(_sc_gather_body, start, npw))
    return run(ids, table)


def _tc_body(sid_ref, x_ref, seg_ref, gamma_ref, beta_ref, w_ref, b_ref,
             *rest):
    out_ref = rest[-1]
    seg = seg_ref[...]
    row0 = seg[0:1, :]
    row1 = seg[1:2, :]
    w = w_ref[...]
    half = M_BLK // 4
    for h in range(4):
        sl = pl.ds(h * half, half)
        x = x_ref[sl, :]
        sid = sid_ref[sl, :].astype(jnp.float32)
        x = x + row0 + sid * (row1 - row0)
        mu = jnp.mean(x, axis=1, keepdims=True)
        xc = x - mu
        var = jnp.mean(xc * xc, axis=1, keepdims=True)
        y = xc * lax.rsqrt(var + EPS) * gamma_ref[...] + beta_ref[...]
        out_ref[sl, :] = (
            jnp.dot(y, w, preferred_element_type=jnp.float32) + b_ref[...]
        )


def _tc_dense_seg(sid, x, seg, gamma, beta, w, b, prev, start, ntok):
    base = start // M_BLK
    in_specs = [
        pl.BlockSpec((M_BLK, 1), lambda i: (i + base, 0)),
        pl.BlockSpec((M_BLK, EMB), lambda i: (i, 0)),
        pl.BlockSpec((2, EMB), lambda i: (0, 0)),
        pl.BlockSpec((1, EMB), lambda i: (0, 0)),
        pl.BlockSpec((1, EMB), lambda i: (0, 0)),
        pl.BlockSpec((EMB, HID), lambda i: (0, 0)),
        pl.BlockSpec((1, HID), lambda i: (0, 0)),
    ]
    args = [sid, x, seg, gamma, beta, w, b]
    aliases = {}
    if prev is not None:
        in_specs.append(pl.BlockSpec(memory_space=pl.MemorySpace.ANY))
        args.append(prev)
        aliases = {7: 0}
    return pl.pallas_call(
        _tc_body,
        grid=(ntok // M_BLK,),
        in_specs=in_specs,
        out_specs=pl.BlockSpec((M_BLK, HID), lambda i: (i + base, 0)),
        out_shape=jax.ShapeDtypeStruct((TOK, HID), jnp.float32),
        input_output_aliases=aliases,
        compiler_params=pltpu.CompilerParams(
            dimension_semantics=("parallel",),
        ),
    )(*args)


def kernel(input_ids, segment_ids, token_table, segment_table, ln_gamma, ln_beta, W, b):
    bsz, seq = input_ids.shape
    ids = input_ids.reshape(TOK)
    sid = segment_ids.reshape(TOK, 1)
    gamma2 = ln_gamma.reshape(1, EMB)
    beta2 = ln_beta.reshape(1, EMB)
    b2 = b.reshape(1, HID)

    gathered = [_sc_gather(s0, n, ids, token_table) for s0, n in SEGMENTS]
    out = None
    for (s0, n), g in zip(SEGMENTS, gathered):
        out = _tc_dense_seg(
            sid, g, segment_table, gamma2, beta2, W, b2, out, s0, n
        )
    return out.reshape(bsz, seq, HID)


# speedup vs baseline: 1.0313x; 1.0313x over previous
import functools

import jax
import jax.numpy as jnp
from jax import lax
from jax.experimental import pallas as pl
from jax.experimental.pallas import tpu as pltpu
from jax.experimental.pallas import tpu_sc as plsc

VOCAB = 100000
EMB = 1024
HID = 4096
EPS = 1e-12

NC = 2
NS = 16
NW = NC * NS

TOK = 8192
CHUNK = 32
M_BLK = 512
SEGMENTS = ((0, 2048), (2048, 6144))


def _sc_gather_body(start, npw, ids_hbm, table_hbm, out_hbm, idx_v,
                    buf0, buf1, buf2, sg0, sg1, sg2, sc0, sc1, sc2):
    nch = npw // CHUNK
    c = lax.axis_index("c")
    s = lax.axis_index("s")
    wid = s * NC + c
    base = wid * npw
    pltpu.sync_copy(ids_hbm.at[pl.ds(start + base, npw)], idx_v)
    bufs = (buf0, buf1, buf2)
    gsem = (sg0, sg1, sg2)
    csem = (sc0, sc1, sc2)

    def start_gather(ch):
        return pltpu.async_copy(
            table_hbm.at[idx_v.at[pl.ds(ch * CHUNK, CHUNK)]],
            bufs[ch % 3], gsem[ch % 3])

    g = [None] * nch
    cp = [None] * nch
    for ch in range(min(2, nch)):
        g[ch] = start_gather(ch)
    for ch in range(nch):
        g[ch].wait()
        if ch >= 1:
            cp[ch - 1].wait()
        if ch + 2 < nch:
            g[ch + 2] = start_gather(ch + 2)
        cp[ch] = pltpu.async_copy(
            bufs[ch % 3], out_hbm.at[pl.ds(base + ch * CHUNK, CHUNK)],
            csem[ch % 3])
    cp[nch - 1].wait()


@functools.partial(jax.jit, static_argnums=(0, 1))
def _sc_gather(start, ntok, ids, table):
    npw = ntok // NW
    mesh = plsc.VectorSubcoreMesh(core_axis_name="c", subcore_axis_name="s")
    run = functools.partial(
        pl.kernel,
        mesh=mesh,
        out_type=jax.ShapeDtypeStruct((ntok, EMB), jnp.float32),
        scratch_types=[
            pltpu.VMEM((npw,), jnp.int32),
            pltpu.VMEM((CHUNK, EMB), jnp.float32),
            pltpu.VMEM((CHUNK, EMB), jnp.float32),
            pltpu.VMEM((CHUNK, EMB), jnp.float32),
            pltpu.SemaphoreType.DMA,
            pltpu.SemaphoreType.DMA,
            pltpu.SemaphoreType.DMA,
            pltpu.SemaphoreType.DMA,
            pltpu.SemaphoreType.DMA,
            pltpu.SemaphoreType.DMA,
        ],
    )(functools.partial(_sc_gather_body, start, npw))
    return run(ids, table)


def _tc_body(sid_ref, x_ref, seg_ref, gamma_ref, beta_ref, w_ref, b_ref,
             *rest):
    out_ref = rest[-1]
    seg = seg_ref[...]
    row0 = seg[0:1, :]
    row1 = seg[1:2, :]
    w = w_ref[...]
    half = M_BLK // 2
    for h in range(2):
        sl = pl.ds(h * half, half)
        x = x_ref[sl, :]
        sid = sid_ref[sl, :].astype(jnp.float32)
        x = x + row0 + sid * (row1 - row0)
        mu = jnp.mean(x, axis=1, keepdims=True)
        xc = x - mu
        var = jnp.mean(xc * xc, axis=1, keepdims=True)
        y = xc * lax.rsqrt(var + EPS) * gamma_ref[...] + beta_ref[...]
        out_ref[sl, :] = (
            jnp.dot(y, w, preferred_element_type=jnp.float32) + b_ref[...]
        )


def _tc_dense_seg(sid, x, seg, gamma, beta, w, b, prev, start, ntok):
    base = start // M_BLK
    in_specs = [
        pl.BlockSpec((M_BLK, 1), lambda i: (i + base, 0)),
        pl.BlockSpec((M_BLK, EMB), lambda i: (i, 0)),
        pl.BlockSpec((2, EMB), lambda i: (0, 0)),
        pl.BlockSpec((1, EMB), lambda i: (0, 0)),
        pl.BlockSpec((1, EMB), lambda i: (0, 0)),
        pl.BlockSpec((EMB, HID), lambda i: (0, 0)),
        pl.BlockSpec((1, HID), lambda i: (0, 0)),
    ]
    args = [sid, x, seg, gamma, beta, w, b]
    aliases = {}
    if prev is not None:
        in_specs.append(pl.BlockSpec(memory_space=pl.MemorySpace.ANY))
        args.append(prev)
        aliases = {7: 0}
    return pl.pallas_call(
        _tc_body,
        grid=(ntok // M_BLK,),
        in_specs=in_specs,
        out_specs=pl.BlockSpec((M_BLK, HID), lambda i: (i + base, 0)),
        out_shape=jax.ShapeDtypeStruct((TOK, HID), jnp.float32),
        input_output_aliases=aliases,
        compiler_params=pltpu.CompilerParams(
            dimension_semantics=("parallel",),
        ),
    )(*args)


def kernel(input_ids, segment_ids, token_table, segment_table, ln_gamma, ln_beta, W, b):
    bsz, seq = input_ids.shape
    ids = input_ids.reshape(TOK)
    sid = segment_ids.reshape(TOK, 1)
    gamma2 = ln_gamma.reshape(1, EMB)
    beta2 = ln_beta.reshape(1, EMB)
    b2 = b.reshape(1, HID)

    gathered = [_sc_gather(s0, n, ids, token_table) for s0, n in SEGMENTS]
    out = None
    for (s0, n), g in zip(SEGMENTS, gathered):
        out = _tc_dense_seg(
            sid, g, segment_table, gamma2, beta2, W, b2, out, s0, n
        )
    return out.reshape(bsz, seq, HID)


# speedup vs baseline: 1.0326x; 1.0013x over previous
import functools

import jax
import jax.numpy as jnp
from jax import lax
from jax.experimental import pallas as pl
from jax.experimental.pallas import tpu as pltpu
from jax.experimental.pallas import tpu_sc as plsc

VOCAB = 100000
EMB = 1024
HID = 4096
EPS = 1e-12

NC = 2
NS = 16
NW = NC * NS

TOK = 8192
CHUNK = 32
M_BLK = 512
SEGMENTS = ((0, 2048), (2048, 6144))


def _sc_gather_body(start, npw, ids_hbm, table_hbm, out_hbm, idx_v,
                    buf0, buf1, buf2, sg0, sg1, sg2, sc0, sc1, sc2):
    nch = npw // CHUNK
    c = lax.axis_index("c")
    s = lax.axis_index("s")
    wid = s * NC + c
    base = wid * npw
    pltpu.sync_copy(ids_hbm.at[pl.ds(start + base, npw)], idx_v)
    bufs = (buf0, buf1, buf2)
    gsem = (sg0, sg1, sg2)
    csem = (sc0, sc1, sc2)

    def start_gather(ch):
        return pltpu.async_copy(
            table_hbm.at[idx_v.at[pl.ds(ch * CHUNK, CHUNK)]],
            bufs[ch % 3], gsem[ch % 3])

    g = [None] * nch
    cp = [None] * nch
    for ch in range(min(2, nch)):
        g[ch] = start_gather(ch)
    for ch in range(nch):
        g[ch].wait()
        if ch >= 1:
            cp[ch - 1].wait()
        if ch + 2 < nch:
            g[ch + 2] = start_gather(ch + 2)
        cp[ch] = pltpu.async_copy(
            bufs[ch % 3], out_hbm.at[pl.ds(base + ch * CHUNK, CHUNK)],
            csem[ch % 3])
    cp[nch - 1].wait()


@functools.partial(jax.jit, static_argnums=(0, 1))
def _sc_gather(start, ntok, ids, table):
    npw = ntok // NW
    mesh = plsc.VectorSubcoreMesh(core_axis_name="c", subcore_axis_name="s")
    run = functools.partial(
        pl.kernel,
        mesh=mesh,
        out_type=jax.ShapeDtypeStruct((ntok, EMB), jnp.float32),
        scratch_types=[
            pltpu.VMEM((npw,), jnp.int32),
            pltpu.VMEM((CHUNK, EMB), jnp.float32),
            pltpu.VMEM((CHUNK, EMB), jnp.float32),
            pltpu.VMEM((CHUNK, EMB), jnp.float32),
            pltpu.SemaphoreType.DMA,
            pltpu.SemaphoreType.DMA,
            pltpu.SemaphoreType.DMA,
            pltpu.SemaphoreType.DMA,
            pltpu.SemaphoreType.DMA,
            pltpu.SemaphoreType.DMA,
        ],
    )(functools.partial(_sc_gather_body, start, npw))
    return run(ids, table)


def _tc_body(sid_ref, x_ref, seg_ref, gamma_ref, beta_ref, w_ref, b_ref,
             *rest):
    out_ref = rest[-1]
    seg = seg_ref[...]
    row0 = seg[0:1, :]
    row1 = seg[1:2, :]
    w = w_ref[...]
    half = M_BLK // 2
    for h in range(2):
        sl = pl.ds(h * half, half)
        x = x_ref[sl, :]
        sid = sid_ref[sl, :].astype(jnp.float32)
        x = x + row0 + sid * (row1 - row0)
        mu = jnp.mean(x, axis=1, keepdims=True)
        xc = x - mu
        var = jnp.mean(xc * xc, axis=1, keepdims=True)
        y = xc * lax.rsqrt(var + EPS) * gamma_ref[...] + beta_ref[...]
        out_ref[sl, :] = (
            jnp.dot(y, w, preferred_element_type=jnp.float32) + b_ref[...]
        )


def _tc_dense_seg(sid, x, seg, gamma, beta, w, b, prev, start, ntok):
    base = start // M_BLK
    in_specs = [
        pl.BlockSpec((M_BLK, 1), lambda i: (i + base, 0)),
        pl.BlockSpec((M_BLK, EMB), lambda i: (i, 0)),
        pl.BlockSpec((2, EMB), lambda i: (0, 0)),
        pl.BlockSpec((1, EMB), lambda i: (0, 0)),
        pl.BlockSpec((1, EMB), lambda i: (0, 0)),
        pl.BlockSpec((EMB, HID), lambda i: (0, 0)),
        pl.BlockSpec((1, HID), lambda i: (0, 0)),
    ]
    args = [sid, x, seg, gamma, beta, w, b]
    aliases = {}
    if prev is not None:
        in_specs.append(pl.BlockSpec(memory_space=pl.MemorySpace.ANY))
        args.append(prev)
        aliases = {7: 0}
    return pl.pallas_call(
        _tc_body,
        grid=(ntok // M_BLK,),
        in_specs=in_specs,
        out_specs=pl.BlockSpec((M_BLK, HID), lambda i: (i + base, 0)),
        out_shape=jax.ShapeDtypeStruct((TOK, HID), jnp.float32),
        input_output_aliases=aliases,
        compiler_params=pltpu.CompilerParams(
            dimension_semantics=("arbitrary",),
        ),
    )(*args)


def kernel(input_ids, segment_ids, token_table, segment_table, ln_gamma, ln_beta, W, b):
    bsz, seq = input_ids.shape
    ids = input_ids.reshape(TOK)
    sid = segment_ids.reshape(TOK, 1)
    gamma2 = ln_gamma.reshape(1, EMB)
    beta2 = ln_beta.reshape(1, EMB)
    b2 = b.reshape(1, HID)

    gathered = [_sc_gather(s0, n, ids, token_table) for s0, n in SEGMENTS]
    out = None
    for (s0, n), g in zip(SEGMENTS, gathered):
        out = _tc_dense_seg(
            sid, g, segment_table, gamma2, beta2, W, b2, out, s0, n
        )
    return out.reshape(bsz, seq, HID)
